# baseline (speedup 1.0000x reference)
import functools

import jax
import jax.numpy as jnp
from jax import lax
from jax.experimental import pallas as pl
from jax.experimental.pallas import tpu as pltpu
from jax.experimental.pallas import tpu_sc as plsc

V, D, B, L = 1000000, 64, 16384, 50

NC, NS = 2, 16
NW = NC * NS
NB = B // NW
SL = 56
NQ = NB * SL + 8
PIPE = 4
GROUPS = NB // 16


def _sc_pack_body(x_hbm, w_hbm, b_hbm, xp_hbm, wb_hbm,
                  x_v, xq_v, w_v, b_v, wb_v):
    wid = lax.axis_index("s") * NC + lax.axis_index("c")
    base = wid * NB

    pltpu.sync_copy(x_hbm.at[pl.ds(base, NB), :], x_v)

    lane = lax.iota(jnp.int32, 16)
    zero = jnp.zeros((16,), jnp.int32)
    tail_cols = jnp.where(lane < 2, 48 + lane, 0)
    tail_mask = lane < 2

    def row_body(r, carry):
        o = r * SL
        for k in range(3):
            xq_v[pl.ds(o + 16 * k, 16)] = x_v[r, pl.ds(16 * k, 16)]
        g = plsc.load_gather(x_v, [jnp.full((16,), r, jnp.int32), tail_cols])
        xq_v[pl.ds(o + 48, 16)] = jnp.where(tail_mask, g, zero)
        return carry

    lax.fori_loop(0, NB, row_body, 0)
    pltpu.sync_copy(xq_v, xp_hbm.at[pl.ds(wid * NQ, NQ)])

    @pl.when(wid == 0)
    def _():
        pltpu.sync_copy(w_hbm, w_v)
        pltpu.sync_copy(b_hbm, b_v)
        for k in range(4):
            wb_v[pl.ds(16 * k, 16)] = w_v[0, pl.ds(16 * k, 16)]
        wb_v[pl.ds(64, 16)] = plsc.load_gather(b_v, [zero])
        pltpu.sync_copy(wb_v, wb_hbm)


@functools.lru_cache(maxsize=1)
def _make_sc_pack():
    mesh = plsc.VectorSubcoreMesh(
        core_axis_name="c", subcore_axis_name="s", num_cores=NC)
    return pl.kernel(
        _sc_pack_body,
        out_type=(jax.ShapeDtypeStruct((NW * NQ,), jnp.int32),
                  jax.ShapeDtypeStruct((80,), jnp.float32)),
        mesh=mesh,
        scratch_types=[
            pltpu.VMEM((NB, L), jnp.int32),
            pltpu.VMEM((NQ,), jnp.int32),
            pltpu.VMEM((1, D), jnp.float32),
            pltpu.VMEM((1,), jnp.float32),
            pltpu.VMEM((80,), jnp.float32),
        ],
        compiler_params=pltpu.CompilerParams(needs_layout_passes=False),
    )


def _sc_pool_body(xp_hbm, emb_hbm, wb_hbm, out_hbm,
                  x_v, buf0, buf1, buf2, buf3, out_v, wb_v,
                  sem0, sem1, sem2, sem3):
    wid = lax.axis_index("s") * NC + lax.axis_index("c")
    base = wid * NB

    pltpu.sync_copy(xp_hbm.at[pl.ds(wid * NQ, NQ)], x_v)
    pltpu.sync_copy(wb_hbm, wb_v)

    w4 = [wb_v[pl.ds(16 * k, 16)] for k in range(4)]
    fcb16 = wb_v[pl.ds(64, 16)]
    lane = lax.iota(jnp.int32, 16)
    lane15 = lane == 15

    bufs = [buf0, buf1, buf2, buf3]
    sems = [sem0, sem1, sem2, sem3]

    def _fire(b, buf, sem):
        pltpu.async_copy(emb_hbm.at[x_v.at[pl.ds(b * SL, SL)]], buf, sem)

    def _drain(buf, sem):
        pltpu.make_async_copy(
            emb_hbm.at[x_v.at[pl.ds(0, SL)]], buf, sem).wait()

    def _process(b, buf):
        acc = [jnp.zeros((16,), jnp.float32) for _ in range(4)]
        for l in range(L):
            for k in range(4):
                acc[k] = acc[k] + buf[l, pl.ds(16 * k, 16)] * w4[k]
        s = (acc[0] + acc[1]) + (acc[2] + acc[3])
        cum = plsc.cumsum(s)
        plsc.store_scatter(out_v, [jnp.full((16,), b, jnp.int32)],
                           cum, mask=lane15)

    for q in range(PIPE):
        _fire(q, bufs[q], sems[q])

    def quad_body(qq, carry):
        b0 = qq * PIPE
        for u in range(PIPE):
            b = b0 + u
            _drain(bufs[u], sems[u])
            _process(b, bufs[u])

            @pl.when(b + PIPE < NB)
            def _():
                _fire(b + PIPE, bufs[u], sems[u])
        return carry

    lax.fori_loop(0, NB // PIPE, quad_body, 0)

    def fin_body(g, carry):
        rows = (g * 16 + lane) * SL
        cnt = jnp.zeros((16,), jnp.float32)
        one = jnp.ones((16,), jnp.float32)
        zero = jnp.zeros((16,), jnp.float32)
        for l in range(L):
            tok = plsc.load_gather(x_v, [rows + l])
            cnt = cnt + jnp.where(tok != 0, one, zero)
        raw = out_v[pl.ds(g * 16, 16)]
        out_v[pl.ds(g * 16, 16)] = raw / jnp.maximum(cnt, one) + fcb16
        return carry

    lax.fori_loop(0, GROUPS, fin_body, 0)
    pltpu.sync_copy(out_v, out_hbm.at[pl.ds(base, NB)])


@functools.lru_cache(maxsize=1)
def _make_sc_pool():
    mesh = plsc.VectorSubcoreMesh(
        core_axis_name="c", subcore_axis_name="s", num_cores=NC)
    return pl.kernel(
        _sc_pool_body,
        out_type=jax.ShapeDtypeStruct((B,), jnp.float32),
        mesh=mesh,
        scratch_types=[
            pltpu.VMEM((NQ,), jnp.int32),
            pltpu.VMEM((SL, D), jnp.float32),
            pltpu.VMEM((SL, D), jnp.float32),
            pltpu.VMEM((SL, D), jnp.float32),
            pltpu.VMEM((SL, D), jnp.float32),
            pltpu.VMEM((NB,), jnp.float32),
            pltpu.VMEM((80,), jnp.float32),
            pltpu.SemaphoreType.DMA,
            pltpu.SemaphoreType.DMA,
            pltpu.SemaphoreType.DMA,
            pltpu.SemaphoreType.DMA,
        ],
        compiler_params=pltpu.CompilerParams(
            needs_layout_passes=False, use_tc_tiling_on_sc=False),
    )


def kernel(x, emb, fc_w, fc_b):
    xp, wb = _make_sc_pack()(x, fc_w, fc_b.astype(jnp.float32))
    return _make_sc_pool()(xp, emb, wb)


# speedup vs baseline: 3.4335x; 3.4335x over previous
import functools

import jax
import jax.numpy as jnp
from jax import lax
from jax.experimental import pallas as pl
from jax.experimental.pallas import tpu as pltpu
from jax.experimental.pallas import tpu_sc as plsc

V, D, B, L = 1000000, 64, 16384, 50

NC, NS = 2, 16
NW = NC * NS
NB = B // NW
SL = 56
NR = NB // 2
PIPE = 4
GROUPS = NB // 16


def _sc_pack_body(x_hbm, w_hbm, b_hbm, xp_hbm, wb_hbm,
                  x_v, xq_v, w_v, b_v, wb_v):
    wid = lax.axis_index("s") * NC + lax.axis_index("c")
    base = wid * NB

    pltpu.sync_copy(x_hbm.at[pl.ds(base, NB), :], x_v)

    lane = lax.iota(jnp.int32, 16)
    zero = jnp.zeros((16,), jnp.int32)
    tail_cols = jnp.where(lane < 2, 48 + lane, lane)

    def row_body(p, carry):
        for sub in range(2):
            r = p * 2 + sub
            cb = sub * SL
            for k in range(3):
                xq_v[p, pl.ds(cb + 16 * k, 16)] = x_v[r, pl.ds(16 * k, 16)]
            tail = plsc.load_gather(
                x_v, [jnp.full((16,), r, jnp.int32), tail_cols])
            xq_v[p, pl.ds(cb + 48, 16)] = tail
        return carry

    lax.fori_loop(0, NR, row_body, 0)
    pltpu.sync_copy(xq_v, xp_hbm.at[pl.ds(wid * NR, NR), :])

    @pl.when(wid == 0)
    def _():
        pltpu.sync_copy(w_hbm, w_v)
        pltpu.sync_copy(b_hbm, b_v)
        for k in range(4):
            wb_v[0, pl.ds(16 * k, 16)] = w_v[0, pl.ds(16 * k, 16)]
            wb_v[1, pl.ds(16 * k, 16)] = plsc.load_gather(b_v, [zero])
            wb_v[2 + k, pl.ds(0, 16)] = jnp.zeros((16,), jnp.float32)
        pltpu.sync_copy(wb_v, wb_hbm)


@functools.lru_cache(maxsize=1)
def _make_sc_pack():
    mesh = plsc.VectorSubcoreMesh(
        core_axis_name="c", subcore_axis_name="s", num_cores=NC)
    return pl.kernel(
        _sc_pack_body,
        out_type=(jax.ShapeDtypeStruct((NW * NR, 128), jnp.int32),
                  jax.ShapeDtypeStruct((8, 128), jnp.float32)),
        mesh=mesh,
        scratch_types=[
            pltpu.VMEM((NB, L), jnp.int32),
            pltpu.VMEM((NR, 128), jnp.int32),
            pltpu.VMEM((1, D), jnp.float32),
            pltpu.VMEM((1,), jnp.float32),
            pltpu.VMEM((8, 128), jnp.float32),
        ],
        compiler_params=pltpu.CompilerParams(needs_layout_passes=False),
    )


def _sc_pool_body(xp_hbm, emb_hbm, wb_hbm, out_hbm,
                  x_v, buf0, buf1, buf2, buf3, out_v, wb_v,
                  sem0, sem1, sem2, sem3):
    wid = lax.axis_index("s") * NC + lax.axis_index("c")

    pltpu.sync_copy(xp_hbm.at[pl.ds(wid * NR, NR), :], x_v)
    pltpu.sync_copy(wb_hbm, wb_v)

    w4 = [wb_v[0, pl.ds(16 * k, 16)] for k in range(4)]
    fcb16 = wb_v[1, pl.ds(0, 16)]
    lane = lax.iota(jnp.int32, 16)
    lane15 = lane == 15

    bufs = [buf0, buf1, buf2, buf3]
    sems = [sem0, sem1, sem2, sem3]

    def _fire(b, buf, sem):
        idx = x_v.at[b // 2, pl.ds((b % 2) * SL, SL)]
        pltpu.async_copy(emb_hbm.at[idx], buf, sem)

    def _drain(buf, sem):
        pltpu.make_async_copy(
            emb_hbm.at[x_v.at[0, pl.ds(0, SL)]], buf, sem).wait()

    def _process(b, buf):
        acc = [jnp.zeros((16,), jnp.float32) for _ in range(4)]
        for l in range(L):
            for k in range(4):
                acc[k] = acc[k] + buf[l, pl.ds(16 * k, 16)] * w4[k]
        s = (acc[0] + acc[1]) + (acc[2] + acc[3])
        cum = plsc.cumsum(s)
        plsc.store_scatter(
            out_v, [jnp.full((16,), b // 128, jnp.int32),
                    jnp.full((16,), b % 128, jnp.int32)], cum, mask=lane15)

    for q in range(PIPE):
        _fire(q, bufs[q], sems[q])

    def quad_body(qq, carry):
        b0 = qq * PIPE
        for u in range(PIPE):
            b = b0 + u
            _drain(bufs[u], sems[u])
            _process(b, bufs[u])

            @pl.when(b + PIPE < NB)
            def _():
                _fire(b + PIPE, bufs[u], sems[u])
        return carry

    lax.fori_loop(0, NB // PIPE, quad_body, 0)

    def fin_body(g, carry):
        bidx = g * 16 + lane
        rows = bidx // 2
        colb = (bidx % 2) * SL
        cnt = jnp.zeros((16,), jnp.float32)
        one = jnp.ones((16,), jnp.float32)
        zero = jnp.zeros((16,), jnp.float32)
        for l in range(L):
            tok = plsc.load_gather(x_v, [rows, colb + l])
            cnt = cnt + jnp.where(tok != 0, one, zero)
        raw = out_v[g // 8, pl.ds((g % 8) * 16, 16)]
        out_v[g // 8, pl.ds((g % 8) * 16, 16)] = (
            raw / jnp.maximum(cnt, one) + fcb16)
        return carry

    lax.fori_loop(0, GROUPS, fin_body, 0)
    pltpu.sync_copy(out_v, out_hbm.at[pl.ds(wid * 4, 4), :])


@functools.lru_cache(maxsize=1)
def _make_sc_pool():
    mesh = plsc.VectorSubcoreMesh(
        core_axis_name="c", subcore_axis_name="s", num_cores=NC)
    return pl.kernel(
        _sc_pool_body,
        out_type=jax.ShapeDtypeStruct((B // 128, 128), jnp.float32),
        mesh=mesh,
        scratch_types=[
            pltpu.VMEM((NR, 128), jnp.int32),
            pltpu.VMEM((SL, D), jnp.float32),
            pltpu.VMEM((SL, D), jnp.float32),
            pltpu.VMEM((SL, D), jnp.float32),
            pltpu.VMEM((SL, D), jnp.float32),
            pltpu.VMEM((4, 128), jnp.float32),
            pltpu.VMEM((8, 128), jnp.float32),
            pltpu.SemaphoreType.DMA,
            pltpu.SemaphoreType.DMA,
            pltpu.SemaphoreType.DMA,
            pltpu.SemaphoreType.DMA,
        ],
        compiler_params=pltpu.CompilerParams(
            needs_layout_passes=False, use_tc_tiling_on_sc=False),
    )


def kernel(x, emb, fc_w, fc_b):
    xp, wb = _make_sc_pack()(x, fc_w, fc_b.astype(jnp.float32))
    out2 = _make_sc_pool()(xp, emb, wb)
    return out2.reshape(B)


# speedup vs baseline: 16.9942x; 4.9495x over previous
import functools

import jax
import jax.numpy as jnp
from jax import lax
from jax.experimental import pallas as pl
from jax.experimental.pallas import tpu as pltpu
from jax.experimental.pallas import tpu_sc as plsc

V, D, B, L = 1000000, 64, 16384, 50

BKW = 32768
FGRID = (V + BKW - 1) // BKW


def _fold_body(embT_ref, wT_ref, p_ref):
    p_ref[...] = jnp.sum(embT_ref[...] * wT_ref[...], axis=0)


def _fold(embT, wT):
    return pl.pallas_call(
        _fold_body,
        grid=(FGRID,),
        in_specs=[
            pl.BlockSpec((D, BKW), lambda i: (0, i)),
            pl.BlockSpec((D, 1), lambda i: (0, 0)),
        ],
        out_specs=pl.BlockSpec((BKW,), lambda i: (i,)),
        out_shape=jax.ShapeDtypeStruct((V,), jnp.float32),
    )(embT, wT)


NC, NS = 2, 16
NW = NC * NS
NB = B // NW
NE = NB * L
CHUNKS = NB // 16


def _sc_pool_body(xf_hbm, p_hbm, fcb_hbm, out_hbm, idx_v, val_v, out_v, fcb_v, sem):
    wid = lax.axis_index("s") * NC + lax.axis_index("c")
    base = wid * NB

    pltpu.sync_copy(xf_hbm.at[pl.ds(wid * NE, NE)], idx_v)
    pltpu.sync_copy(fcb_hbm, fcb_v)
    pltpu.async_copy(p_hbm.at[idx_v], val_v, sem).wait()

    fcb16 = fcb_v[...]
    lane = lax.iota(jnp.int32, 16)
    lane_off = lane * L

    def chunk_body(c, carry):
        bvec = c * (16 * L) + lane_off
        acc = jnp.zeros((16,), jnp.float32)
        cnt = jnp.zeros((16,), jnp.float32)
        one = jnp.ones((16,), jnp.float32)
        zero = jnp.zeros((16,), jnp.float32)
        for l in range(L):
            g = bvec + l
            acc = acc + plsc.load_gather(val_v, [g])
            tok = plsc.load_gather(idx_v, [g])
            cnt = cnt + jnp.where(tok != 0, one, zero)
        out_v[pl.ds(c * 16, 16)] = acc / jnp.maximum(cnt, one) + fcb16
        return carry

    lax.fori_loop(0, CHUNKS, chunk_body, 0)
    pltpu.sync_copy(out_v, out_hbm.at[pl.ds(base, NB)])


@functools.lru_cache(maxsize=1)
def _make_sc_pool():
    mesh = plsc.VectorSubcoreMesh(
        core_axis_name="c", subcore_axis_name="s", num_cores=NC)
    return pl.kernel(
        _sc_pool_body,
        out_type=jax.ShapeDtypeStruct((B,), jnp.float32),
        mesh=mesh,
        scratch_types=[
            pltpu.VMEM((NE,), jnp.int32),
            pltpu.VMEM((NE,), jnp.float32),
            pltpu.VMEM((NB,), jnp.float32),
            pltpu.VMEM((16,), jnp.float32),
            pltpu.SemaphoreType.DMA,
        ],
        compiler_params=pltpu.CompilerParams(
            needs_layout_passes=False, use_tc_tiling_on_sc=False),
    )


def kernel(x, emb, fc_w, fc_b):
    embT = emb.T
    wT = fc_w.reshape(D, 1)
    p = _fold(embT, wT)
    xf = x.reshape(B * L)
    fcb16 = jnp.broadcast_to(fc_b.astype(jnp.float32), (16,))
    return _make_sc_pool()(xf, p, fcb16)
